# speedup vs baseline: 12.3676x; 12.3676x over previous
import functools

import jax
import jax.numpy as jnp
from jax import lax
from jax.experimental import pallas as pl
from jax.experimental.pallas import tpu as pltpu
from jax.experimental.pallas import tpu_sc as plsc

N = 10000
D = 128
E = 320000
NC, NS = 2, 16
NW = NC * NS
K = 79
EPW = K * 128
EPAD = NW * EPW
NP = N + 16
ACC = 10240
RPT = ACC // NS

_mesh = plsc.VectorSubcoreMesh(core_axis_name="c", subcore_axis_name="s")


@functools.partial(
    pl.kernel,
    out_type=jax.ShapeDtypeStruct((NC, ACC), jnp.float32),
    mesh=_mesh,
    scratch_types=[
        pltpu.VMEM((K, 128), jnp.int32),
        pltpu.VMEM((128,), jnp.float32),
        pltpu.VMEM((RPT,), jnp.float32),
        pltpu.VMEM_SHARED((ACC,), jnp.float32),
    ],
)
def _deg_kernel(dst_hbm, ones_hbm, zer_hbm, out_hbm, idx_v, ones_v, z_v, acc_s):
    c = lax.axis_index("c")
    s = lax.axis_index("s")
    wid = s * NC + c
    pltpu.sync_copy(ones_hbm, ones_v)
    pltpu.sync_copy(zer_hbm, z_v)
    pltpu.sync_copy(z_v, acc_s.at[pl.ds(s * RPT, RPT)])
    plsc.subcore_barrier()
    pltpu.sync_copy(dst_hbm.at[wid], idx_v)

    def body(j, carry):
        pltpu.sync_copy(ones_v, acc_s.at[idx_v.at[j]], add=True)
        return carry

    lax.fori_loop(0, K, body, 0)
    plsc.subcore_barrier()
    pltpu.sync_copy(acc_s.at[pl.ds(s * RPT, RPT)], out_hbm.at[c, pl.ds(s * RPT, RPT)])


@functools.partial(
    pl.kernel,
    out_type=jax.ShapeDtypeStruct((NC, ACC, D), jnp.float32),
    mesh=_mesh,
    scratch_types=[
        pltpu.VMEM((K, 128), jnp.int32),
        pltpu.VMEM((K, 128), jnp.int32),
        pltpu.VMEM((128, D), jnp.float32),
        pltpu.VMEM((64, D), jnp.float32),
        pltpu.VMEM_SHARED((ACC, D), jnp.float32),
        pltpu.SemaphoreType.DMA,
    ],
)
def _agg_kernel(g_hbm, src_hbm, dst_hbm, zer_hbm, out_hbm,
                src_v, dst_v, rows_v, z_v, acc_s, sem):
    c = lax.axis_index("c")
    s = lax.axis_index("s")
    wid = s * NC + c
    pltpu.sync_copy(zer_hbm, z_v)
    for t in range(RPT // 64):
        pltpu.sync_copy(z_v, acc_s.at[pl.ds(s * RPT + t * 64, 64), :])
    plsc.subcore_barrier()
    pltpu.sync_copy(src_hbm.at[wid], src_v)
    pltpu.sync_copy(dst_hbm.at[wid], dst_v)

    def body(j, carry):
        pltpu.async_copy(g_hbm.at[src_v.at[j]], rows_v, sem).wait()
        pltpu.sync_copy(rows_v, acc_s.at[dst_v.at[j]], add=True)
        return carry

    lax.fori_loop(0, K, body, 0)
    plsc.subcore_barrier()
    pltpu.sync_copy(acc_s.at[pl.ds(s * RPT, RPT), :],
                    out_hbm.at[c, pl.ds(s * RPT, RPT), :])


def _mm1_body(x_ref, w_ref, deg_ref, g_ref):
    dis = lax.rsqrt(deg_ref[0] + deg_ref[1] + 1.0)
    g_ref[...] = jnp.dot(x_ref[...], w_ref[...],
                         preferred_element_type=jnp.float32) * dis


def _mid_body(p_ref, g1_ref, deg_ref, b_ref, w_ref, g2_ref):
    dis = lax.rsqrt(deg_ref[0] + deg_ref[1] + 1.0)
    acc = p_ref[0] + p_ref[1] + g1_ref[...]
    y = jnp.maximum(acc * dis + b_ref[...], 0.0)
    g2_ref[...] = jnp.dot(y, w_ref[...],
                          preferred_element_type=jnp.float32) * dis


def _out_body(p_ref, g2_ref, deg_ref, b_ref, o_ref):
    dis = lax.rsqrt(deg_ref[0] + deg_ref[1] + 1.0)
    acc = p_ref[0] + p_ref[1] + g2_ref[...]
    o_ref[...] = acc * dis + b_ref[...]


_mm1 = pl.pallas_call(
    _mm1_body, out_shape=jax.ShapeDtypeStruct((NP, D), jnp.float32))
_mid = pl.pallas_call(
    _mid_body, out_shape=jax.ShapeDtypeStruct((NP, D), jnp.float32))
_out = pl.pallas_call(
    _out_body, out_shape=jax.ShapeDtypeStruct((NP, D), jnp.float32))


def kernel(x, edge_index, W1, b1, W2, b2):
    ei = edge_index.astype(jnp.int32)
    pad = jnp.full((EPAD - E,), N, dtype=jnp.int32)
    src = jnp.concatenate([ei[0], pad]).reshape(NW, K, 128)
    dst = jnp.concatenate([ei[1], pad]).reshape(NW, K, 128)

    ones128 = jnp.ones((128,), jnp.float32)
    zer_row = jnp.zeros((RPT,), jnp.float32)
    zer_2d = jnp.zeros((64, D), jnp.float32)

    degp = _deg_kernel(dst, ones128, zer_row)
    deg = degp[:, :NP, None]

    x_pad = jnp.concatenate([x, jnp.zeros((NP - N, D), x.dtype)])
    g1 = _mm1(x_pad, W1, deg)

    p1 = _agg_kernel(g1, src, dst, zer_2d)[:, :NP, :]
    g2 = _mid(p1, g1, deg, b1.reshape(1, D), W2)

    p2 = _agg_kernel(g2, src, dst, zer_2d)[:, :NP, :]
    out = _out(p2, g2, deg, b2.reshape(1, D))
    return out[:N]


# speedup vs baseline: 15.2016x; 1.2292x over previous
import functools

import jax
import jax.numpy as jnp
from jax import lax
from jax.experimental import pallas as pl
from jax.experimental.pallas import tpu as pltpu
from jax.experimental.pallas import tpu_sc as plsc

N = 10000
D = 128
E = 320000
NC, NS = 2, 16
NW = NC * NS
K = 79
EPW = K * 128
EPAD = NW * EPW
NP = N + 16
ACC = 10240
RPT = ACC // NS

_mesh = plsc.VectorSubcoreMesh(core_axis_name="c", subcore_axis_name="s")


@functools.partial(
    pl.kernel,
    out_type=jax.ShapeDtypeStruct((NC, ACC), jnp.float32),
    mesh=_mesh,
    scratch_types=[
        pltpu.VMEM((K, 128), jnp.int32),
        pltpu.VMEM((128,), jnp.float32),
        pltpu.VMEM((RPT,), jnp.float32),
        pltpu.VMEM_SHARED((ACC,), jnp.float32),
    ],
)
def _deg_kernel(dst_hbm, ones_hbm, zer_hbm, out_hbm, idx_v, ones_v, z_v, acc_s):
    c = lax.axis_index("c")
    s = lax.axis_index("s")
    wid = s * NC + c
    pltpu.sync_copy(ones_hbm, ones_v)
    pltpu.sync_copy(zer_hbm, z_v)
    pltpu.sync_copy(z_v, acc_s.at[pl.ds(s * RPT, RPT)])
    plsc.subcore_barrier()
    pltpu.sync_copy(dst_hbm.at[wid], idx_v)

    def body(j, carry):
        pltpu.sync_copy(ones_v, acc_s.at[idx_v.at[j]], add=True)
        return carry

    lax.fori_loop(0, K, body, 0)
    plsc.subcore_barrier()
    pltpu.sync_copy(acc_s.at[pl.ds(s * RPT, RPT)], out_hbm.at[c, pl.ds(s * RPT, RPT)])


@functools.partial(
    pl.kernel,
    out_type=jax.ShapeDtypeStruct((NC, ACC, D), jnp.float32),
    mesh=_mesh,
    scratch_types=[
        pltpu.VMEM((3, 2, 128), jnp.int32),
        pltpu.VMEM((2, 128, D), jnp.float32),
        pltpu.VMEM_SHARED((ACC, D), jnp.float32),
        pltpu.SemaphoreType.DMA((3,)),
        pltpu.SemaphoreType.DMA((2,)),
    ],
)
def _agg_kernel(g_hbm, idx_hbm, zer_hbm, out_hbm,
                idx_v, rows_v, acc_s, isem, sem):
    c = lax.axis_index("c")
    s = lax.axis_index("s")
    wid = s * NC + c
    pltpu.sync_copy(zer_hbm, acc_s.at[pl.ds(s * RPT, RPT), :])
    plsc.subcore_barrier()

    pltpu.async_copy(idx_hbm.at[wid, 0], idx_v.at[0], isem.at[0])
    pltpu.async_copy(idx_hbm.at[wid, 1], idx_v.at[1], isem.at[1])
    pltpu.make_async_copy(idx_hbm.at[wid, 0], idx_v.at[0], isem.at[0]).wait()
    pltpu.async_copy(g_hbm.at[idx_v.at[0, 0]], rows_v.at[0], sem.at[0])

    def body(j, carry):
        p = lax.rem(j, 2)
        r = lax.rem(j, 3)
        r2 = lax.rem(j + 2, 3)

        @pl.when(j + 2 < K)
        def _():
            pltpu.async_copy(idx_hbm.at[wid, j + 2], idx_v.at[r2],
                             isem.at[r2])

        pltpu.make_async_copy(g_hbm.at[idx_v.at[r, 0]], rows_v.at[p],
                              sem.at[p]).wait()

        @pl.when(j + 1 < K)
        def _():
            q = lax.rem(j + 1, 2)
            r1 = lax.rem(j + 1, 3)
            pltpu.make_async_copy(idx_hbm.at[wid, j + 1], idx_v.at[r1],
                                  isem.at[r1]).wait()
            pltpu.async_copy(g_hbm.at[idx_v.at[r1, 0]], rows_v.at[q],
                             sem.at[q])

        pltpu.sync_copy(rows_v.at[p], acc_s.at[idx_v.at[r, 1]], add=True)
        return carry

    lax.fori_loop(0, K, body, 0)
    plsc.subcore_barrier()
    pltpu.sync_copy(acc_s.at[pl.ds(s * RPT, RPT), :],
                    out_hbm.at[c, pl.ds(s * RPT, RPT), :])


def _mm1_body(x_ref, w_ref, deg_ref, g_ref):
    dis = lax.rsqrt(deg_ref[0] + deg_ref[1] + 1.0)
    g_ref[...] = jnp.dot(x_ref[...], w_ref[...],
                         preferred_element_type=jnp.float32) * dis


def _mid_body(p_ref, g1_ref, deg_ref, b_ref, w_ref, g2_ref):
    dis = lax.rsqrt(deg_ref[0] + deg_ref[1] + 1.0)
    acc = p_ref[0] + p_ref[1] + g1_ref[...]
    y = jnp.maximum(acc * dis + b_ref[...], 0.0)
    g2_ref[...] = jnp.dot(y, w_ref[...],
                          preferred_element_type=jnp.float32) * dis


def _out_body(p_ref, g2_ref, deg_ref, b_ref, o_ref):
    dis = lax.rsqrt(deg_ref[0] + deg_ref[1] + 1.0)
    acc = p_ref[0] + p_ref[1] + g2_ref[...]
    o_ref[...] = acc * dis + b_ref[...]


_mm1 = pl.pallas_call(
    _mm1_body, out_shape=jax.ShapeDtypeStruct((NP, D), jnp.float32))
_mid = pl.pallas_call(
    _mid_body, out_shape=jax.ShapeDtypeStruct((NP, D), jnp.float32))
_out = pl.pallas_call(
    _out_body, out_shape=jax.ShapeDtypeStruct((NP, D), jnp.float32))


def kernel(x, edge_index, W1, b1, W2, b2):
    ei = edge_index.astype(jnp.int32)
    pad = jnp.full((EPAD - E,), N, dtype=jnp.int32)
    src = jnp.concatenate([ei[0], pad]).reshape(NW, K, 1, 128)
    dst = jnp.concatenate([ei[1], pad]).reshape(NW, K, 1, 128)
    idx = jnp.concatenate([src, dst], axis=2)

    ones128 = jnp.ones((128,), jnp.float32)
    zer_row = jnp.zeros((RPT,), jnp.float32)
    zer_2d = jnp.zeros((RPT, D), jnp.float32)

    degp = _deg_kernel(dst.reshape(NW, K, 128), ones128, zer_row)
    deg = degp[:, :NP, None]

    x_pad = jnp.concatenate([x, jnp.zeros((NP - N, D), x.dtype)])
    g1 = _mm1(x_pad, W1, deg)

    p1 = _agg_kernel(g1, idx, zer_2d)[:, :NP, :]
    g2 = _mid(p1, g1, deg, b1.reshape(1, D), W2)

    p2 = _agg_kernel(g2, idx, zer_2d)[:, :NP, :]
    out = _out(p2, g2, deg, b2.reshape(1, D))
    return out[:N]
